# Initial kernel scaffold; baseline (speedup 1.0000x reference)
import jax
import jax.numpy as jnp
from jax.experimental import pallas as pl


def kernel(x, edge_index, edge_attr, W_i, W_h, W_o, b_o):
    raise NotImplementedError("write your pallas kernel here")


# speedup vs baseline: 2.3543x; 2.3543x over previous
import functools

import jax
import jax.numpy as jnp
from jax import lax
from jax.experimental import pallas as pl
from jax.experimental.pallas import tpu as pltpu
from jax.experimental.pallas import tpu_sc as plsc

NC = 2
NS = 16
CH = 80


def _sc_mesh():
  return plsc.VectorSubcoreMesh(
      core_axis_name="c", subcore_axis_name="s",
      num_cores=NC, num_subcores=NS)


def _make_sc_segment(E, N, H, with_gather):
  HH = H // NC
  rows_per_tile = E // NS
  nchunk = rows_per_tile // CH
  npad = ((N + 8 * NS - 1) // (8 * NS)) * (8 * NS)
  zrows = npad // NS

  out_shape = (E, H) if with_gather else (N, H)

  scratch = [
      pltpu.VMEM((nchunk, CH), jnp.int32),
      pltpu.VMEM((nchunk, CH), jnp.int32),
      pltpu.VMEM((CH, HH), jnp.float32),
      pltpu.VMEM_SHARED((npad, HH), jnp.float32),
  ]

  @functools.partial(
      pl.kernel,
      out_type=jax.ShapeDtypeStruct(out_shape, jnp.float32),
      mesh=_sc_mesh(),
      scratch_types=scratch,
  )
  def body(msg_hbm, dst_hbm, src_hbm, zero_hbm, out_hbm,
           dst_v, src_v, buf, acc):
    c = lax.axis_index("c")
    s = lax.axis_index("s")
    col0 = c * HH
    pltpu.sync_copy(zero_hbm, acc.at[pl.ds(s * zrows, zrows)])
    pltpu.sync_copy(dst_hbm.at[s], dst_v)
    if with_gather:
      pltpu.sync_copy(src_hbm.at[s], src_v)
    plsc.subcore_barrier()

    row0 = s * rows_per_tile

    def scatter_body(k, carry):
      r0 = row0 + k * CH
      pltpu.sync_copy(msg_hbm.at[pl.ds(r0, CH), pl.ds(col0, HH)], buf)
      pltpu.sync_copy(buf, acc.at[dst_v.at[k]], add=True)
      return carry

    lax.fori_loop(0, nchunk, scatter_body, 0)
    plsc.subcore_barrier()

    if with_gather:
      def gather_body(k, carry):
        r0 = row0 + k * CH
        pltpu.sync_copy(acc.at[src_v.at[k]], buf)
        pltpu.sync_copy(buf, out_hbm.at[pl.ds(r0, CH), pl.ds(col0, HH)])
        return carry

      lax.fori_loop(0, nchunk, gather_body, 0)
    else:
      tail = N - (NS - 1) * zrows

      @pl.when(s < NS - 1)
      def _():
        pltpu.sync_copy(
            acc.at[pl.ds(s * zrows, zrows)],
            out_hbm.at[pl.ds(s * zrows, zrows), pl.ds(col0, HH)])

      @pl.when(s == NS - 1)
      def _():
        pltpu.sync_copy(
            acc.at[pl.ds((NS - 1) * zrows, tail)],
            out_hbm.at[pl.ds((NS - 1) * zrows, tail), pl.ds(col0, HH)])

  return body


def _tc_init(E, H, d_edge):
  RB = 2000
  grid = (E // RB,)

  def body(ea_ref, wiT_ref, li_ref, msg_ref):
    li = jnp.dot(ea_ref[...], wiT_ref[...],
                 preferred_element_type=jnp.float32)
    li_ref[...] = li
    msg_ref[...] = jnp.maximum(li, 0.0)

  return pl.pallas_call(
      body,
      grid=grid,
      in_specs=[
          pl.BlockSpec((RB, d_edge), lambda i: (i, 0)),
          pl.BlockSpec((d_edge, H), lambda i: (0, 0)),
      ],
      out_specs=[
          pl.BlockSpec((RB, H), lambda i: (i, 0)),
          pl.BlockSpec((RB, H), lambda i: (i, 0)),
      ],
      out_shape=[
          jax.ShapeDtypeStruct((E, H), jnp.float32),
          jax.ShapeDtypeStruct((E, H), jnp.float32),
      ],
  )


def _tc_update(E, H):
  RB = 1600
  nb = E // RB
  shift = (E // 2) // RB

  def body(li_ref, g_ref, mrev_ref, whT_ref, out_ref):
    xv = g_ref[...] - mrev_ref[...]
    acc = jnp.dot(xv, whT_ref[...], preferred_element_type=jnp.float32)
    out_ref[...] = jnp.maximum(li_ref[...] + acc, 0.0)

  return pl.pallas_call(
      body,
      grid=(nb,),
      in_specs=[
          pl.BlockSpec((RB, H), lambda i: (i, 0)),
          pl.BlockSpec((RB, H), lambda i: (i, 0)),
          pl.BlockSpec((RB, H), lambda i: ((i + shift) % nb, 0)),
          pl.BlockSpec((H, H), lambda i: (0, 0)),
      ],
      out_specs=pl.BlockSpec((RB, H), lambda i: (i, 0)),
      out_shape=jax.ShapeDtypeStruct((E, H), jnp.float32),
  )


def _tc_final(N, H, d_node):
  RB = 1000
  nb = N // RB

  def body(x_ref, nn_ref, woxT_ref, wohT_ref, bo_ref, out_ref):
    i = pl.program_id(0)
    a = (jnp.dot(x_ref[...], woxT_ref[...],
                 preferred_element_type=jnp.float32)
         + jnp.dot(nn_ref[...], wohT_ref[...],
                   preferred_element_type=jnp.float32)
         + bo_ref[...])
    part = jnp.sum(jnp.maximum(a, 0.0), axis=0, keepdims=True)

    @pl.when(i == 0)
    def _():
      out_ref[...] = jnp.zeros_like(out_ref)

    out_ref[...] += part

    @pl.when(i == nb - 1)
    def _():
      out_ref[...] = out_ref[...] * (1.0 / N)

  return pl.pallas_call(
      body,
      grid=(nb,),
      in_specs=[
          pl.BlockSpec((RB, d_node), lambda i: (i, 0)),
          pl.BlockSpec((RB, H), lambda i: (i, 0)),
          pl.BlockSpec((d_node, H), lambda i: (0, 0)),
          pl.BlockSpec((H, H), lambda i: (0, 0)),
          pl.BlockSpec((1, H), lambda i: (0, 0)),
      ],
      out_specs=pl.BlockSpec((1, H), lambda i: (0, 0)),
      out_shape=jax.ShapeDtypeStruct((1, H), jnp.float32),
  )


def kernel(x, edge_index, edge_attr, W_i, W_h, W_o, b_o):
  N, d_node = x.shape
  E, d_edge = edge_attr.shape
  H = W_i.shape[0]
  depth = 4

  src = edge_index[0]
  dst = edge_index[1]
  nchunk = E // NS // CH
  dst3d = dst.reshape(NS, nchunk, CH)
  src3d = src.reshape(NS, nchunk, CH)
  npad = ((N + 8 * NS - 1) // (8 * NS)) * (8 * NS)
  zeros = jnp.zeros((npad // NS, H // NC), jnp.float32)

  wiT = W_i.T
  whT = W_h.T
  woxT = W_o[:, :d_node].T
  wohT = W_o[:, d_node:].T
  bo2d = b_o.reshape(1, H)

  sc_gather = _make_sc_segment(E, N, H, with_gather=True)
  sc_reduce = _make_sc_segment(E, N, H, with_gather=False)

  line_input, msg = _tc_init(E, H, d_edge)(edge_attr, wiT)
  update = _tc_update(E, H)
  for _ in range(depth - 1):
    gath = sc_gather(msg, dst3d, src3d, zeros)
    msg = update(line_input, gath, msg, whT)

  node_nei = sc_reduce(msg, dst3d, src3d, zeros)
  return _tc_final(N, H, d_node)(x, node_nei, woxT, wohT, bo2d)


# speedup vs baseline: 3.1114x; 1.3216x over previous
import functools

import jax
import jax.numpy as jnp
from jax import lax
from jax.experimental import pallas as pl
from jax.experimental.pallas import tpu as pltpu
from jax.experimental.pallas import tpu_sc as plsc

NC = 2
NS = 16
CH = 80


def _sc_mesh():
  return plsc.VectorSubcoreMesh(
      core_axis_name="c", subcore_axis_name="s",
      num_cores=NC, num_subcores=NS)


def _make_sc_segment(E, N, H, with_gather):
  HH = H // NC
  rows_per_tile = E // NS
  nchunk = rows_per_tile // CH
  npad = ((N + 8 * NS - 1) // (8 * NS)) * (8 * NS)
  zrows = npad // NS

  out_shape = (E, H) if with_gather else (N, H)

  assert nchunk % 2 == 1

  scratch = [
      pltpu.VMEM((nchunk, CH), jnp.int32),
      pltpu.VMEM((CH, HH), jnp.float32),
      pltpu.VMEM((CH, HH), jnp.float32),
      pltpu.VMEM_SHARED((npad, HH), jnp.float32),
      pltpu.SemaphoreType.DMA,
      pltpu.SemaphoreType.DMA,
  ]

  @functools.partial(
      pl.kernel,
      out_type=jax.ShapeDtypeStruct(out_shape, jnp.float32),
      mesh=_sc_mesh(),
      scratch_types=scratch,
  )
  def body(msg_hbm, dst_hbm, src_hbm, zero_hbm, out_hbm,
           idx_v, buf0, buf1, acc, sem0, sem1):
    c = lax.axis_index("c")
    s = lax.axis_index("s")
    col0 = c * HH
    pltpu.sync_copy(zero_hbm, acc.at[pl.ds(s * zrows, zrows)])
    pltpu.sync_copy(dst_hbm.at[s], idx_v)
    plsc.subcore_barrier()

    row0 = s * rows_per_tile

    def in_slice(k):
      return msg_hbm.at[pl.ds(row0 + k * CH, CH), pl.ds(col0, HH)]

    def drain_in(buf, sem):
      pltpu.make_async_copy(in_slice(0), buf, sem).wait()

    pltpu.async_copy(in_slice(0), buf0, sem0)
    pltpu.async_copy(in_slice(1), buf1, sem1)

    def scatter_pair(p, carry):
      k0 = 2 * p

      def one(k, buf, sem):
        drain_in(buf, sem)
        pltpu.sync_copy(buf, acc.at[idx_v.at[k]], add=True)

        @pl.when(k + 2 < nchunk)
        def _():
          pltpu.async_copy(in_slice(k + 2), buf, sem)

      one(k0, buf0, sem0)
      one(k0 + 1, buf1, sem1)
      return carry

    lax.fori_loop(0, nchunk // 2, scatter_pair, 0)
    drain_in(buf0, sem0)
    pltpu.sync_copy(buf0, acc.at[idx_v.at[nchunk - 1]], add=True)
    plsc.subcore_barrier()

    if with_gather:
      pltpu.sync_copy(src_hbm.at[s], idx_v)
      def out_slice(k):
        return out_hbm.at[pl.ds(row0 + k * CH, CH), pl.ds(col0, HH)]

      def drain_out(buf, sem):
        pltpu.make_async_copy(buf, out_slice(0), sem).wait()

      def gather_pair(p, carry):
        k0 = 2 * p

        def one(k, buf, sem):
          @pl.when(k >= 2)
          def _():
            drain_out(buf, sem)

          pltpu.sync_copy(acc.at[idx_v.at[k]], buf)
          pltpu.async_copy(buf, out_slice(k), sem)

        one(k0, buf0, sem0)
        one(k0 + 1, buf1, sem1)
        return carry

      lax.fori_loop(0, nchunk // 2, gather_pair, 0)
      drain_out(buf0, sem0)
      pltpu.sync_copy(acc.at[idx_v.at[nchunk - 1]], buf0)
      pltpu.async_copy(buf0, out_slice(nchunk - 1), sem0)
      drain_out(buf0, sem0)
      drain_out(buf1, sem1)
    else:
      tail = N - (NS - 1) * zrows

      @pl.when(s < NS - 1)
      def _():
        pltpu.sync_copy(
            acc.at[pl.ds(s * zrows, zrows)],
            out_hbm.at[pl.ds(s * zrows, zrows), pl.ds(col0, HH)])

      @pl.when(s == NS - 1)
      def _():
        pltpu.sync_copy(
            acc.at[pl.ds((NS - 1) * zrows, tail)],
            out_hbm.at[pl.ds((NS - 1) * zrows, tail), pl.ds(col0, HH)])

  return body


def _tc_init(E, H, d_edge):
  RB = 2000
  grid = (E // RB,)

  def body(ea_ref, wiT_ref, li_ref, msg_ref):
    li = jnp.dot(ea_ref[...], wiT_ref[...],
                 preferred_element_type=jnp.float32)
    li_ref[...] = li
    msg_ref[...] = jnp.maximum(li, 0.0)

  return pl.pallas_call(
      body,
      grid=grid,
      in_specs=[
          pl.BlockSpec((RB, d_edge), lambda i: (i, 0)),
          pl.BlockSpec((d_edge, H), lambda i: (0, 0)),
      ],
      out_specs=[
          pl.BlockSpec((RB, H), lambda i: (i, 0)),
          pl.BlockSpec((RB, H), lambda i: (i, 0)),
      ],
      out_shape=[
          jax.ShapeDtypeStruct((E, H), jnp.float32),
          jax.ShapeDtypeStruct((E, H), jnp.float32),
      ],
  )


def _tc_update(E, H):
  RB = 1600
  nb = E // RB
  shift = (E // 2) // RB

  def body(li_ref, g_ref, mrev_ref, whT_ref, out_ref):
    xv = g_ref[...] - mrev_ref[...]
    acc = jnp.dot(xv, whT_ref[...], preferred_element_type=jnp.float32)
    out_ref[...] = jnp.maximum(li_ref[...] + acc, 0.0)

  return pl.pallas_call(
      body,
      grid=(nb,),
      in_specs=[
          pl.BlockSpec((RB, H), lambda i: (i, 0)),
          pl.BlockSpec((RB, H), lambda i: (i, 0)),
          pl.BlockSpec((RB, H), lambda i: ((i + shift) % nb, 0)),
          pl.BlockSpec((H, H), lambda i: (0, 0)),
      ],
      out_specs=pl.BlockSpec((RB, H), lambda i: (i, 0)),
      out_shape=jax.ShapeDtypeStruct((E, H), jnp.float32),
  )


def _tc_final(N, H, d_node):
  RB = 1000
  nb = N // RB

  def body(x_ref, nn_ref, woxT_ref, wohT_ref, bo_ref, out_ref):
    i = pl.program_id(0)
    a = (jnp.dot(x_ref[...], woxT_ref[...],
                 preferred_element_type=jnp.float32)
         + jnp.dot(nn_ref[...], wohT_ref[...],
                   preferred_element_type=jnp.float32)
         + bo_ref[...])
    part = jnp.sum(jnp.maximum(a, 0.0), axis=0, keepdims=True)

    @pl.when(i == 0)
    def _():
      out_ref[...] = jnp.zeros_like(out_ref)

    out_ref[...] += part

    @pl.when(i == nb - 1)
    def _():
      out_ref[...] = out_ref[...] * (1.0 / N)

  return pl.pallas_call(
      body,
      grid=(nb,),
      in_specs=[
          pl.BlockSpec((RB, d_node), lambda i: (i, 0)),
          pl.BlockSpec((RB, H), lambda i: (i, 0)),
          pl.BlockSpec((d_node, H), lambda i: (0, 0)),
          pl.BlockSpec((H, H), lambda i: (0, 0)),
          pl.BlockSpec((1, H), lambda i: (0, 0)),
      ],
      out_specs=pl.BlockSpec((1, H), lambda i: (0, 0)),
      out_shape=jax.ShapeDtypeStruct((1, H), jnp.float32),
  )


def kernel(x, edge_index, edge_attr, W_i, W_h, W_o, b_o):
  N, d_node = x.shape
  E, d_edge = edge_attr.shape
  H = W_i.shape[0]
  depth = 4

  src = edge_index[0]
  dst = edge_index[1]
  nchunk = E // NS // CH
  dst3d = dst.reshape(NS, nchunk, CH)
  src3d = src.reshape(NS, nchunk, CH)
  npad = ((N + 8 * NS - 1) // (8 * NS)) * (8 * NS)
  zeros = jnp.zeros((npad // NS, H // NC), jnp.float32)

  wiT = W_i.T
  whT = W_h.T
  woxT = W_o[:, :d_node].T
  wohT = W_o[:, d_node:].T
  bo2d = b_o.reshape(1, H)

  sc_gather = _make_sc_segment(E, N, H, with_gather=True)
  sc_reduce = _make_sc_segment(E, N, H, with_gather=False)

  line_input, msg = _tc_init(E, H, d_edge)(edge_attr, wiT)
  update = _tc_update(E, H)
  for _ in range(depth - 1):
    gath = sc_gather(msg, dst3d, src3d, zeros)
    msg = update(line_input, gath, msg, whT)

  node_nei = sc_reduce(msg, dst3d, src3d, zeros)
  return _tc_final(N, H, d_node)(x, node_nei, woxT, wohT, bo2d)


# speedup vs baseline: 3.2947x; 1.0589x over previous
import functools

import jax
import jax.numpy as jnp
from jax import lax
from jax.experimental import pallas as pl
from jax.experimental.pallas import tpu as pltpu
from jax.experimental.pallas import tpu_sc as plsc

NC = 2
NS = 16
CH = 80


def _sc_mesh():
  return plsc.VectorSubcoreMesh(
      core_axis_name="c", subcore_axis_name="s",
      num_cores=NC, num_subcores=NS)


def _make_sc_segment(E, N, H, with_gather):
  HH = H // NC
  rows_per_tile = E // NS
  nchunk = rows_per_tile // CH
  npad = ((N + 8 * NS - 1) // (8 * NS)) * (8 * NS)
  zrows = npad // NS

  out_shape = (E, H) if with_gather else (N, H)

  assert nchunk % 2 == 1

  scratch = [
      pltpu.VMEM((nchunk, CH), jnp.int32),
      pltpu.VMEM((CH, HH), jnp.float32),
      pltpu.VMEM((CH, HH), jnp.float32),
      pltpu.VMEM_SHARED((npad, HH), jnp.float32),
      pltpu.SemaphoreType.DMA,
      pltpu.SemaphoreType.DMA,
  ]

  @functools.partial(
      pl.kernel,
      out_type=jax.ShapeDtypeStruct(out_shape, jnp.float32),
      mesh=_sc_mesh(),
      scratch_types=scratch,
  )
  def body(msg_hbm, dst_hbm, src_hbm, zero_hbm, out_hbm,
           idx_v, buf0, buf1, acc, sem0, sem1):
    c = lax.axis_index("c")
    s = lax.axis_index("s")
    col0 = c * HH
    pltpu.sync_copy(zero_hbm, acc.at[pl.ds(s * zrows, zrows)])
    pltpu.sync_copy(dst_hbm.at[s], idx_v)
    plsc.subcore_barrier()

    row0 = s * rows_per_tile

    def in_slice(k):
      return msg_hbm.at[pl.ds(row0 + k * CH, CH), pl.ds(col0, HH)]

    def drain_in(buf, sem):
      pltpu.make_async_copy(in_slice(0), buf, sem).wait()

    pltpu.async_copy(in_slice(0), buf0, sem0)
    pltpu.async_copy(in_slice(1), buf1, sem1)

    def scatter_pair(p, carry):
      k0 = 2 * p

      def one(k, buf, sem):
        drain_in(buf, sem)
        pltpu.sync_copy(buf, acc.at[idx_v.at[k]], add=True)

        @pl.when(k + 2 < nchunk)
        def _():
          pltpu.async_copy(in_slice(k + 2), buf, sem)

      one(k0, buf0, sem0)
      one(k0 + 1, buf1, sem1)
      return carry

    lax.fori_loop(0, nchunk // 2, scatter_pair, 0)
    drain_in(buf0, sem0)
    pltpu.sync_copy(buf0, acc.at[idx_v.at[nchunk - 1]], add=True)
    plsc.subcore_barrier()

    if with_gather:
      pltpu.sync_copy(src_hbm.at[s], idx_v)
      def out_slice(k):
        return out_hbm.at[pl.ds(row0 + k * CH, CH), pl.ds(col0, HH)]

      def drain_out(buf, sem):
        pltpu.make_async_copy(buf, out_slice(0), sem).wait()

      def gather_pair(p, carry):
        k0 = 2 * p

        def one(k, buf, sem):
          @pl.when(k >= 2)
          def _():
            drain_out(buf, sem)

          pltpu.sync_copy(acc.at[idx_v.at[k]], buf)
          pltpu.async_copy(buf, out_slice(k), sem)

        one(k0, buf0, sem0)
        one(k0 + 1, buf1, sem1)
        return carry

      lax.fori_loop(0, nchunk // 2, gather_pair, 0)
      drain_out(buf0, sem0)
      pltpu.sync_copy(acc.at[idx_v.at[nchunk - 1]], buf0)
      pltpu.async_copy(buf0, out_slice(nchunk - 1), sem0)
      drain_out(buf0, sem0)
      drain_out(buf1, sem1)
    else:
      tail = N - (NS - 1) * zrows

      @pl.when(s < NS - 1)
      def _():
        pltpu.sync_copy(
            acc.at[pl.ds(s * zrows, zrows)],
            out_hbm.at[pl.ds(s * zrows, zrows), pl.ds(col0, HH)])

      @pl.when(s == NS - 1)
      def _():
        pltpu.sync_copy(
            acc.at[pl.ds((NS - 1) * zrows, tail)],
            out_hbm.at[pl.ds((NS - 1) * zrows, tail), pl.ds(col0, HH)])

  return body


def _tc_init(E, H, d_edge):
  RB = 2000
  grid = (E // RB,)

  def body(ea_ref, wiT_ref, li_ref, msg_ref):
    li = jnp.dot(ea_ref[...], wiT_ref[...],
                 preferred_element_type=jnp.float32)
    li_ref[...] = li.astype(jnp.bfloat16)
    msg_ref[...] = jnp.maximum(li, 0.0)

  return pl.pallas_call(
      body,
      grid=grid,
      in_specs=[
          pl.BlockSpec((RB, d_edge), lambda i: (i, 0)),
          pl.BlockSpec((d_edge, H), lambda i: (0, 0)),
      ],
      out_specs=[
          pl.BlockSpec((RB, H), lambda i: (i, 0)),
          pl.BlockSpec((RB, H), lambda i: (i, 0)),
      ],
      out_shape=[
          jax.ShapeDtypeStruct((E, H), jnp.bfloat16),
          jax.ShapeDtypeStruct((E, H), jnp.float32),
      ],
  )


def _tc_update(E, H):
  RB = 1600
  nb = E // RB
  shift = (E // 2) // RB

  def body(li_ref, g_ref, mrev_ref, whT_ref, out_ref):
    xv = g_ref[...] - mrev_ref[...]
    acc = jnp.dot(xv, whT_ref[...], preferred_element_type=jnp.float32)
    out_ref[...] = jnp.maximum(li_ref[...].astype(jnp.float32) + acc, 0.0)

  return pl.pallas_call(
      body,
      grid=(nb,),
      in_specs=[
          pl.BlockSpec((RB, H), lambda i: (i, 0)),
          pl.BlockSpec((RB, H), lambda i: (i, 0)),
          pl.BlockSpec((RB, H), lambda i: ((i + shift) % nb, 0)),
          pl.BlockSpec((H, H), lambda i: (0, 0)),
      ],
      out_specs=pl.BlockSpec((RB, H), lambda i: (i, 0)),
      out_shape=jax.ShapeDtypeStruct((E, H), jnp.float32),
  )


def _tc_final(N, H, d_node):
  RB = 1000
  nb = N // RB

  def body(x_ref, nn_ref, woxT_ref, wohT_ref, bo_ref, out_ref):
    i = pl.program_id(0)
    a = (jnp.dot(x_ref[...], woxT_ref[...],
                 preferred_element_type=jnp.float32)
         + jnp.dot(nn_ref[...], wohT_ref[...],
                   preferred_element_type=jnp.float32)
         + bo_ref[...])
    part = jnp.sum(jnp.maximum(a, 0.0), axis=0, keepdims=True)

    @pl.when(i == 0)
    def _():
      out_ref[...] = jnp.zeros_like(out_ref)

    out_ref[...] += part

    @pl.when(i == nb - 1)
    def _():
      out_ref[...] = out_ref[...] * (1.0 / N)

  return pl.pallas_call(
      body,
      grid=(nb,),
      in_specs=[
          pl.BlockSpec((RB, d_node), lambda i: (i, 0)),
          pl.BlockSpec((RB, H), lambda i: (i, 0)),
          pl.BlockSpec((d_node, H), lambda i: (0, 0)),
          pl.BlockSpec((H, H), lambda i: (0, 0)),
          pl.BlockSpec((1, H), lambda i: (0, 0)),
      ],
      out_specs=pl.BlockSpec((1, H), lambda i: (0, 0)),
      out_shape=jax.ShapeDtypeStruct((1, H), jnp.float32),
  )


def kernel(x, edge_index, edge_attr, W_i, W_h, W_o, b_o):
  N, d_node = x.shape
  E, d_edge = edge_attr.shape
  H = W_i.shape[0]
  depth = 4

  src = edge_index[0]
  dst = edge_index[1]
  nchunk = E // NS // CH
  dst3d = dst.reshape(NS, nchunk, CH)
  src3d = src.reshape(NS, nchunk, CH)
  npad = ((N + 8 * NS - 1) // (8 * NS)) * (8 * NS)
  zeros = jnp.zeros((npad // NS, H // NC), jnp.float32)

  wiT = W_i.T
  whT = W_h.T
  woxT = W_o[:, :d_node].T
  wohT = W_o[:, d_node:].T
  bo2d = b_o.reshape(1, H)

  sc_gather = _make_sc_segment(E, N, H, with_gather=True)
  sc_reduce = _make_sc_segment(E, N, H, with_gather=False)

  line_input, msg = _tc_init(E, H, d_edge)(edge_attr, wiT)
  update = _tc_update(E, H)
  for _ in range(depth - 1):
    gath = sc_gather(msg, dst3d, src3d, zeros)
    msg = update(line_input, gath, msg, whT)

  node_nei = sc_reduce(msg, dst3d, src3d, zeros)
  return _tc_final(N, H, d_node)(x, node_nei, woxT, wohT, bo2d)
